# baseline (speedup 1.0000x reference)
import functools

import jax
import jax.numpy as jnp
from jax import lax
from jax.experimental import pallas as pl
from jax.experimental.pallas import tpu as pltpu
from jax.experimental.pallas import tpu_sc as plsc

B = 256
BC = 32
NCHUNK = B // BC
NC, NS = 2, 16
NW = NC * NS
UNROLL = 16


@functools.lru_cache(maxsize=None)
def _make_layer(I, O):
    npw = O // NW
    mesh = plsc.VectorSubcoreMesh(core_axis_name="c", subcore_axis_name="s",
                                  num_cores=NC, num_subcores=NS)

    @functools.partial(
        pl.kernel,
        out_type=jax.ShapeDtypeStruct((NCHUNK, O, BC), jnp.float32),
        mesh=mesh,
        compiler_params=pltpu.CompilerParams(use_tc_tiling_on_sc=False),
        scratch_types=[
            pltpu.VMEM((2 * I, BC), jnp.float32),
            pltpu.VMEM((npw, I), jnp.float32),
            pltpu.VMEM((16,), jnp.int32),
            pltpu.VMEM((16,), jnp.float32),
            pltpu.VMEM((16,), jnp.float32),
            pltpu.VMEM((npw, BC), jnp.float32),
        ],
    )
    def layer(z_hbm, m_hbm, base_hbm, p_hbm, q_hbm, out_hbm,
              z_v, m_v, b_v, p_v, q_v, o_v):
        c = lax.axis_index("c")
        s = lax.axis_index("s")
        w = s * NC + c
        pltpu.sync_copy(m_hbm.at[pl.ds(w * npw, npw), :], m_v)
        pltpu.sync_copy(base_hbm.at[w], b_v)
        pltpu.sync_copy(p_hbm.at[w], p_v)
        pltpu.sync_copy(q_hbm.at[w], q_v)

        bvec = b_v[...]
        pvec = p_v[...]
        qvec = q_v[...]

        def chunk(ci, carry):
            pltpu.sync_copy(z_hbm.at[ci], z_v)
            for o in range(npw):
                base = bvec[o]
                p = pvec[o]
                q = qvec[o]

                def body(iu, accs, o=o, base=base):
                    a0, a1 = accs
                    i0 = iu * UNROLL
                    mvec = m_v[o, pl.ds(i0, UNROLL)]
                    for u in range(UNROLL):
                        m = mvec[u]
                        z0 = z_v[base + i0 + u, pl.ds(0, 16)]
                        z1 = z_v[base + i0 + u, pl.ds(16, 16)]
                        a0 = jnp.maximum(a0, m * z0)
                        a1 = jnp.maximum(a1, m * z1)
                    return a0, a1

                zero = jnp.zeros((16,), jnp.float32)
                a0, a1 = lax.fori_loop(0, I // UNROLL, body, (zero, zero))
                o_v[o, pl.ds(0, 16)] = p + q * a0
                o_v[o, pl.ds(16, 16)] = p + q * a1
            pltpu.sync_copy(o_v, out_hbm.at[ci, pl.ds(w * npw, npw), :])
            return carry

        lax.fori_loop(0, NCHUNK, chunk, 0)

    return layer


def _node_params(ntc, g):
    logits = ntc + g
    y_soft = jax.nn.softmax(logits, axis=-1)
    amax = jnp.argmax(logits, axis=-1)
    y_hard = jax.nn.one_hot(amax, 2, dtype=logits.dtype)
    sel = y_soft + (y_hard - y_soft)
    is_max = amax == 1
    base = jnp.where(is_max, 0, 1).astype(jnp.int32)
    p = jnp.where(is_max, 0.0, sel[:, 0])
    q = jnp.where(is_max, sel[:, 1], -sel[:, 0])
    return base, p, q


def _pack_worker(a, npw):
    a = a.reshape(NW, npw)
    pad = jnp.zeros((NW, 16 - npw), a.dtype)
    return jnp.concatenate([a, pad], axis=1)


def kernel(x, ntc1, ntc2, g1, g2, noedge1, noedge2):
    b1, p1, q1 = _node_params(ntc1, g1)
    b2, p2, q2 = _node_params(ntc2, g2)
    b1 = b1 * 512
    b2 = b2 * 256
    m1 = jnp.logical_not(noedge1).astype(jnp.float32)
    m2 = jnp.logical_not(noedge2).astype(jnp.float32)

    zt = jnp.concatenate([x.T, (1.0 - x).T], axis=0)
    z1 = zt.reshape(1024, NCHUNK, BC).transpose(1, 0, 2)

    h = _make_layer(512, 256)(z1, m1, _pack_worker(b1, 256 // NW),
                _pack_worker(p1, 256 // NW), _pack_worker(q1, 256 // NW))
    z2 = jnp.concatenate([h, 1.0 - h], axis=1)

    out = _make_layer(256, 512)(z2, m2, _pack_worker(b2, 512 // NW),
                  _pack_worker(p2, 512 // NW), _pack_worker(q2, 512 // NW))
    return out.transpose(1, 0, 2).reshape(512, B).T
